# Initial kernel scaffold; baseline (speedup 1.0000x reference)
import functools

import jax
import jax.numpy as jnp
from jax import lax
from jax.experimental import pallas as pl
from jax.experimental.pallas import tpu as pltpu
from jax.experimental.pallas import tpu_sc as plsc

NUM_EMBEDDINGS = 1000000
D = 32
PADDING_IDX = 0

_INFO = plsc.get_sparse_core_info()
NC = _INFO.num_cores
NS = _INFO.num_subcores
L = _INFO.num_lanes
NW = NC * NS

B = 16384 * 200
PER_W = B // NW
C = 1600
G = PER_W // C


@functools.partial(
    pl.kernel,
    out_type=jax.ShapeDtypeStruct((B, D), jnp.float32),
    mesh=plsc.VectorSubcoreMesh(core_axis_name="c", subcore_axis_name="s"),
    scratch_types=[
        pltpu.VMEM((C,), jnp.int32),
        pltpu.VMEM((C, D), jnp.float32),
        pltpu.SemaphoreType.DMA,
    ],
)
def _gather_kernel(idx_hbm, table_hbm, out_hbm, idx_v, rows_v, sem):
    wid = lax.axis_index("s") * NC + lax.axis_index("c")
    w_base = wid * PER_W

    def chunk_body(g, _):
        base = w_base + g * C
        pltpu.sync_copy(idx_hbm.at[pl.ds(base, C)], idx_v)
        pltpu.async_copy(table_hbm.at[idx_v], rows_v, sem).wait()

        def fix_body(i, _):
            v = idx_v[pl.ds(i * L, L)]
            mn = jnp.min(v)

            @pl.when(mn == PADDING_IDX)
            def _repair():
                m = v == PADDING_IDX
                row16 = i * L + lax.iota(jnp.int32, L)
                zeros = jnp.zeros((L,), jnp.float32)
                for k in range(D):
                    col = jnp.full((L,), k, jnp.int32)
                    plsc.store_scatter(rows_v, [row16, col], zeros, mask=m)

            return 0

        lax.fori_loop(0, C // L, fix_body, 0)
        pltpu.sync_copy(rows_v, out_hbm.at[pl.ds(base, C)])
        return 0

    lax.fori_loop(0, G, chunk_body, 0)


def kernel(idx, embeddings):
    idx_flat = idx.reshape(-1).astype(jnp.int32)
    out = _gather_kernel(idx_flat, embeddings)
    return out.reshape(idx.shape + (D,))


# speedup vs baseline: 4.7158x; 4.7158x over previous
import functools

import jax
import jax.numpy as jnp
from jax import lax
from jax.experimental import pallas as pl
from jax.experimental.pallas import tpu as pltpu
from jax.experimental.pallas import tpu_sc as plsc

NUM_EMBEDDINGS = 1000000
D = 32
PADDING_IDX = 0

_INFO = plsc.get_sparse_core_info()
NC = _INFO.num_cores
NS = _INFO.num_subcores
L = _INFO.num_lanes
NW = NC * NS

B = 16384 * 200
PER_W = B // NW
C = 1600
G = PER_W // C

_IGNORE = 7


@functools.partial(
    pl.kernel,
    out_type=(
        jax.ShapeDtypeStruct((B, D), jnp.float32),
        jax.ShapeDtypeStruct((NC, D), jnp.float32),
    ),
    mesh=plsc.VectorSubcoreMesh(core_axis_name="c", subcore_axis_name="s"),
    scratch_types=[
        pltpu.VMEM((C,), jnp.int32),
        pltpu.VMEM((C,), jnp.int32),
        pltpu.VMEM((C, D), jnp.float32),
        pltpu.SemaphoreType.DMA,
    ],
    compiler_params=pltpu.CompilerParams(use_tc_tiling_on_sc=False),
)
def _gather_kernel(idx_hbm, table_hbm, out_hbm, zeros_hbm, idx_v, zsel_v,
                   rows_v, sem):
    cid = lax.axis_index("c")
    sid = lax.axis_index("s")
    wid = sid * NC + cid
    w_base = wid * PER_W

    @pl.when(sid == 0)
    def _init_zero_row():
        zvec = jnp.zeros((L,), jnp.float32)
        for k in range(D // L):
            rows_v[0, pl.ds(k * L, L)] = zvec
        pltpu.sync_copy(rows_v.at[pl.ds(0, 1)], zeros_hbm.at[pl.ds(cid, 1)])

    plsc.subcore_barrier()

    def chunk_body(g, _):
        base = w_base + g * C
        pltpu.sync_copy(idx_hbm.at[pl.ds(base, C)], idx_v)

        def zsel_body(i, _):
            v = idx_v[pl.ds(i * L, L)]
            zsel_v[pl.ds(i * L, L)] = jnp.where(
                v == PADDING_IDX, cid, _IGNORE
            ).astype(jnp.int32)
            return 0

        lax.fori_loop(0, C // L, zsel_body, 0)

        gat = pltpu.async_copy(
            table_hbm.at[plsc.Indices(idx_v, ignored_value=PADDING_IDX)],
            rows_v, sem)
        zfill = pltpu.async_copy(
            zeros_hbm.at[plsc.Indices(zsel_v, ignored_value=_IGNORE)],
            rows_v, sem)
        gat.wait()
        zfill.wait()

        pltpu.sync_copy(rows_v, out_hbm.at[pl.ds(base, C)])
        return 0

    lax.fori_loop(0, G, chunk_body, 0)


def kernel(idx, embeddings):
    idx_flat = idx.reshape(-1).astype(jnp.int32)
    out, _ = _gather_kernel(idx_flat, embeddings)
    return out.reshape(idx.shape + (D,))


# speedup vs baseline: 5.0843x; 1.0781x over previous
import functools

import jax
import jax.numpy as jnp
from jax import lax
from jax.experimental import pallas as pl
from jax.experimental.pallas import tpu as pltpu
from jax.experimental.pallas import tpu_sc as plsc

NUM_EMBEDDINGS = 1000000
D = 32
PADDING_IDX = 0

_INFO = plsc.get_sparse_core_info()
NC = _INFO.num_cores
NS = _INFO.num_subcores
L = _INFO.num_lanes
NW = NC * NS

B = 16384 * 200
PER_W = B // NW
C = 1600
G = PER_W // C
NI = G // 2

_IGNORE = 7


@functools.partial(
    pl.kernel,
    out_type=(
        jax.ShapeDtypeStruct((B, D), jnp.float32),
        jax.ShapeDtypeStruct((NC, D), jnp.float32),
    ),
    mesh=plsc.VectorSubcoreMesh(core_axis_name="c", subcore_axis_name="s"),
    scratch_types=[
        pltpu.VMEM((C,), jnp.int32),
        pltpu.VMEM((C,), jnp.int32),
        pltpu.VMEM((C,), jnp.int32),
        pltpu.VMEM((C,), jnp.int32),
        pltpu.VMEM((C, D), jnp.float32),
        pltpu.VMEM((C, D), jnp.float32),
        pltpu.SemaphoreType.DMA,
        pltpu.SemaphoreType.DMA,
        pltpu.SemaphoreType.DMA,
        pltpu.SemaphoreType.DMA,
        pltpu.SemaphoreType.DMA,
        pltpu.SemaphoreType.DMA,
        pltpu.SemaphoreType.DMA,
        pltpu.SemaphoreType.DMA,
    ],
    compiler_params=pltpu.CompilerParams(use_tc_tiling_on_sc=False),
)
def _gather_kernel(idx_hbm, table_hbm, out_hbm, zeros_hbm,
                   idx0, idx1, zsel0, zsel1, rows0, rows1,
                   si0, si1, sa0, sa1, sb0, sb1, so0, so1):
    cid = lax.axis_index("c")
    sid = lax.axis_index("s")
    wid = sid * NC + cid
    w_base = wid * PER_W

    def idx_start(g, idx_v, sem):
        pltpu.async_copy(idx_hbm.at[pl.ds(w_base + g * C, C)], idx_v, sem)

    def idx_wait(g, idx_v, sem):
        pltpu.make_async_copy(
            idx_hbm.at[pl.ds(w_base + g * C, C)], idx_v, sem).wait()

    def build_zsel(idx_v, zsel_v):
        def body(i, _):
            v = idx_v[pl.ds(i * L, L)]
            zsel_v[pl.ds(i * L, L)] = jnp.where(
                v == PADDING_IDX, cid, _IGNORE
            ).astype(jnp.int32)
            return 0
        lax.fori_loop(0, C // L, body, 0)

    def gathers_start(idx_v, zsel_v, rows_v, sa, sb):
        pltpu.async_copy(
            table_hbm.at[plsc.Indices(idx_v, ignored_value=PADDING_IDX)],
            rows_v, sa)
        pltpu.async_copy(
            zeros_hbm.at[plsc.Indices(zsel_v, ignored_value=_IGNORE)],
            rows_v, sb)

    def gathers_wait(idx_v, zsel_v, rows_v, sa, sb):
        pltpu.make_async_copy(
            table_hbm.at[plsc.Indices(idx_v, ignored_value=PADDING_IDX)],
            rows_v, sa).wait()
        pltpu.make_async_copy(
            zeros_hbm.at[plsc.Indices(zsel_v, ignored_value=_IGNORE)],
            rows_v, sb).wait()

    def out_start(g, rows_v, sem):
        pltpu.async_copy(rows_v, out_hbm.at[pl.ds(w_base + g * C, C)], sem)

    def out_wait(g, rows_v, sem):
        pltpu.make_async_copy(
            rows_v, out_hbm.at[pl.ds(w_base + g * C, C)], sem).wait()

    idx_start(0, idx0, si0)

    @pl.when(sid == 0)
    def _init_zero_row():
        zvec = jnp.zeros((L,), jnp.float32)
        for k in range(D // L):
            rows0[0, pl.ds(k * L, L)] = zvec
        pltpu.sync_copy(rows0.at[pl.ds(0, 1)], zeros_hbm.at[pl.ds(cid, 1)])

    plsc.subcore_barrier()

    def body(i, _):
        g0 = 2 * i
        g1 = g0 + 1

        @pl.when(i > 0)
        def _():
            out_wait(g0 - 2, rows0, so0)
        idx_wait(g0, idx0, si0)
        build_zsel(idx0, zsel0)
        gathers_start(idx0, zsel0, rows0, sa0, sb0)

        @pl.when(i > 0)
        def _():
            gathers_wait(idx1, zsel1, rows1, sa1, sb1)
            out_start(g0 - 1, rows1, so1)
        idx_start(g1, idx1, si1)

        @pl.when(i > 0)
        def _():
            out_wait(g1 - 2, rows1, so1)
        idx_wait(g1, idx1, si1)
        build_zsel(idx1, zsel1)
        gathers_start(idx1, zsel1, rows1, sa1, sb1)

        gathers_wait(idx0, zsel0, rows0, sa0, sb0)
        out_start(g0, rows0, so0)

        @pl.when(i < NI - 1)
        def _():
            idx_start(g1 + 1, idx0, si0)
        return 0

    lax.fori_loop(0, NI, body, 0)

    gathers_wait(idx1, zsel1, rows1, sa1, sb1)
    out_start(G - 1, rows1, so1)
    out_wait(G - 2, rows0, so0)
    out_wait(G - 1, rows1, so1)


def kernel(idx, embeddings):
    idx_flat = idx.reshape(-1).astype(jnp.int32)
    out, _ = _gather_kernel(idx_flat, embeddings)
    return out.reshape(idx.shape + (D,))
